# baseline (speedup 1.0000x reference)
import jax
import jax.numpy as jnp
from jax.experimental import pallas as pl
from jax.experimental.pallas import tpu as pltpu

_NCHUNKS = 2


def _body(Iv, d_h, zl_h, o1_h, o2_h, dv, zv, o1v, o2v, insem, outsem):
    B = Iv.shape[0]
    r = B // _NCHUNKS
    ins = []
    for c in range(_NCHUNKS):
        sl = pl.ds(c * r, r)
        ins.append((
            pltpu.async_copy(d_h.at[sl], dv.at[sl], insem.at[c]),
            pltpu.async_copy(zl_h.at[sl], zv.at[sl], insem.at[c]),
        ))
    outs = []
    for c in range(_NCHUNKS):
        sl = pl.ds(c * r, r)
        for cp in ins[c]:
            cp.wait()
        m = Iv[sl].astype(jnp.float32)
        zlI = zv[sl] * m
        nd = -(dv[sl] * m)
        o1v[sl] = zlI * jnp.maximum(nd, 0.0)
        o2v[sl] = zlI * jnp.minimum(nd, 0.0)
        outs.append(pltpu.async_copy(o1v.at[sl], o1_h.at[sl], outsem.at[c]))
        outs.append(pltpu.async_copy(o2v.at[sl], o2_h.at[sl], outsem.at[c]))
    for cp in outs:
        cp.wait()


def kernel(I, d, zl):
    B, n = d.shape
    out = jax.ShapeDtypeStruct((B, n), jnp.float32)
    spec = pl.BlockSpec(memory_space=pl.ANY)
    return pl.pallas_call(
        _body,
        out_shape=(out, out),
        in_specs=[pl.BlockSpec(memory_space=pltpu.VMEM), spec, spec],
        out_specs=(spec, spec),
        scratch_shapes=[
            pltpu.VMEM((B, n), jnp.float32),
            pltpu.VMEM((B, n), jnp.float32),
            pltpu.VMEM((B, n), jnp.float32),
            pltpu.VMEM((B, n), jnp.float32),
            pltpu.SemaphoreType.DMA((_NCHUNKS,)),
            pltpu.SemaphoreType.DMA((_NCHUNKS,)),
        ],
        compiler_params=pltpu.CompilerParams(
            disable_bounds_checks=True,
            disable_semaphore_checks=True,
        ),
    )(I, d, zl)


# speedup vs baseline: 1.1886x; 1.1886x over previous
import jax
import jax.numpy as jnp
from jax.experimental import pallas as pl
from jax.experimental.pallas import tpu as pltpu


def _body(I_ref, d_ref, zl_ref, o_zl_ref, o_zu_ref):
    m = I_ref[...].astype(jnp.float32)
    zlI = zl_ref[...] * m
    nd = -(d_ref[...] * m)
    o_zl_ref[...] = zlI * jnp.maximum(nd, 0.0)
    o_zu_ref[...] = zlI * jnp.minimum(nd, 0.0)


def kernel(I, d, zl):
    B, n = d.shape
    out = jax.ShapeDtypeStruct((B, n), jnp.float32)
    spec = pl.BlockSpec(memory_space=pltpu.VMEM)
    return pl.pallas_call(
        _body,
        out_shape=(out, out),
        in_specs=[spec, spec, spec],
        out_specs=(spec, spec),
        compiler_params=pltpu.CompilerParams(
            disable_bounds_checks=True,
            disable_semaphore_checks=True,
        ),
    )(I, d, zl)
